# speedup vs baseline: 5.0681x; 5.0681x over previous
import functools

import jax
import jax.numpy as jnp
from jax import lax
from jax.experimental import pallas as pl
from jax.experimental.pallas import tpu as pltpu
from jax.experimental.pallas import tpu_sc as plsc

N_NODES = 10000
NFEAT = 128
BN_EPS = 1e-5

NC = 2
NS = 16
NW = NC * NS
G = 128
ACC = 10240
ROWS_PER_TILE = ACC // NS


def _make_sc_agg(n_chunks, with_cnt):
    mesh = plsc.VectorSubcoreMesh(core_axis_name="c", subcore_axis_name="s")
    out_type = [jax.ShapeDtypeStruct((NC, ACC, NFEAT), jnp.float32)]
    scratch = [
        pltpu.VMEM((n_chunks, G), jnp.int32),
        pltpu.VMEM((n_chunks, G), jnp.int32),
        pltpu.VMEM((G, NFEAT), jnp.float32),
        pltpu.VMEM_SHARED((ACC, NFEAT), jnp.float32),
        pltpu.SemaphoreType.DMA,
    ]
    if with_cnt:
        out_type.append(jax.ShapeDtypeStruct((NC, ACC), jnp.float32))
        scratch.append(pltpu.VMEM((G,), jnp.float32))
        scratch.append(pltpu.VMEM_SHARED((ACC,), jnp.float32))

    @functools.partial(pl.kernel, mesh=mesh, out_type=out_type,
                       scratch_types=scratch)
    def sc_agg(x_hbm, src_hbm, dst_hbm, zeros2d, zeros1d, acc_out, *rest):
        if with_cnt:
            cnt_out, src_v, dst_v, rows_v, acc_sh, sem, ones_v, cnt_sh = rest
        else:
            src_v, dst_v, rows_v, acc_sh, sem = rest
        c = lax.axis_index("c")
        s = lax.axis_index("s")
        wid = c * NS + s
        base = s * ROWS_PER_TILE
        pltpu.sync_copy(zeros2d.at[pl.ds(base, ROWS_PER_TILE)],
                        acc_sh.at[pl.ds(base, ROWS_PER_TILE)])
        if with_cnt:
            pltpu.sync_copy(zeros1d.at[pl.ds(base, ROWS_PER_TILE)],
                            cnt_sh.at[pl.ds(base, ROWS_PER_TILE)])
            for i in range(G // 16):
                ones_v[pl.ds(i * 16, 16)] = jnp.full((16,), 1.0, jnp.float32)
        pltpu.sync_copy(src_hbm.at[wid], src_v)
        pltpu.sync_copy(dst_hbm.at[wid], dst_v)
        plsc.subcore_barrier()

        def body(j, carry):
            pltpu.async_copy(x_hbm.at[src_v.at[j]], rows_v, sem).wait()
            pltpu.sync_copy(rows_v, acc_sh.at[dst_v.at[j]], add=True)
            if with_cnt:
                pltpu.sync_copy(ones_v, cnt_sh.at[dst_v.at[j]], add=True)
            return carry

        lax.fori_loop(0, n_chunks, body, 0)
        plsc.subcore_barrier()
        pltpu.sync_copy(acc_sh.at[pl.ds(base, ROWS_PER_TILE)],
                        acc_out.at[c, pl.ds(base, ROWS_PER_TILE)])
        if with_cnt:
            pltpu.sync_copy(cnt_sh.at[pl.ds(base, ROWS_PER_TILE)],
                            cnt_out.at[c, pl.ds(base, ROWS_PER_TILE)])

    return sc_agg


def _make_tc_dense(with_post):
    BLK = 1000
    grid = N_NODES // BLK

    def body(part_ref, cnt_ref, x_ref, wl_ref, bl_ref, wr_ref,
             g_ref, b_ref, m_ref, v_ref, out_ref):
        agg = part_ref[0] + part_ref[1]
        cnt = cnt_ref[0] + cnt_ref[1]
        agg = agg / jnp.maximum(cnt, 1.0)
        out = (jnp.dot(agg, wl_ref[...], preferred_element_type=jnp.float32)
               + bl_ref[0][None, :]
               + jnp.dot(x_ref[...], wr_ref[...],
                         preferred_element_type=jnp.float32))
        nrm = jnp.sqrt(jnp.sum(out * out, axis=1, keepdims=True))
        out = out / jnp.maximum(nrm, 1e-12)
        if with_post:
            out = jnp.maximum(out, 0.0)
            scale = g_ref[0][None, :] / jnp.sqrt(v_ref[0][None, :] + BN_EPS)
            out = (out - m_ref[0][None, :]) * scale + b_ref[0][None, :]
        out_ref[...] = out

    vec = pl.BlockSpec((1, NFEAT), lambda i: (0, 0))
    mat = pl.BlockSpec((NFEAT, NFEAT), lambda i: (0, 0))
    return pl.pallas_call(
        body,
        grid=(grid,),
        in_specs=[
            pl.BlockSpec((NC, BLK, NFEAT), lambda i: (0, i, 0)),
            pl.BlockSpec((NC, BLK, 1), lambda i: (0, i, 0)),
            pl.BlockSpec((BLK, NFEAT), lambda i: (i, 0)),
            mat, vec, mat, vec, vec, vec, vec,
        ],
        out_specs=pl.BlockSpec((BLK, NFEAT), lambda i: (i, 0)),
        out_shape=jax.ShapeDtypeStruct((N_NODES, NFEAT), jnp.float32),
    )


def kernel(x, edge_index, W1_l, b1_l, W1_r, bn_gamma, bn_beta, bn_mean,
           bn_var, W2_l, b2_l, W2_r):
    n_edges = edge_index.shape[1]
    e_pad = ((n_edges + NW * G - 1) // (NW * G)) * (NW * G)
    n_chunks = e_pad // (NW * G)
    pad = e_pad - n_edges
    src = edge_index[0].astype(jnp.int32)
    dst = edge_index[1].astype(jnp.int32)
    if pad:
        src = jnp.concatenate([src, jnp.zeros((pad,), jnp.int32)])
        dst = jnp.concatenate(
            [dst, N_NODES + (jnp.arange(pad, dtype=jnp.int32) % (ACC - N_NODES))])
    src = src.reshape(NW, n_chunks, G)
    dst = dst.reshape(NW, n_chunks, G)
    zeros2d = jnp.zeros((ACC, NFEAT), jnp.float32)
    zeros1d = jnp.zeros((ACC,), jnp.float32)

    sc_agg1 = _make_sc_agg(n_chunks, with_cnt=True)
    sc_agg2 = _make_sc_agg(n_chunks, with_cnt=False)
    tc1 = _make_tc_dense(with_post=True)
    tc2 = _make_tc_dense(with_post=False)

    vec = lambda a: a.reshape(1, NFEAT)
    part1, cnt = sc_agg1(x, src, dst, zeros2d, zeros1d)
    cnt3 = cnt.reshape(NC, ACC, 1)
    h = tc1(part1, cnt3, x, W1_l, vec(b1_l), W1_r,
            vec(bn_gamma), vec(bn_beta), vec(bn_mean), vec(bn_var))
    part2 = sc_agg2(h, src, dst, zeros2d, zeros1d)
    if isinstance(part2, (list, tuple)):
        part2 = part2[0]
    out = tc2(part2, cnt3, h, W2_l, vec(b2_l), W2_r,
              vec(bn_gamma), vec(bn_beta), vec(bn_mean), vec(bn_var))
    return out
